# baseline (speedup 1.0000x reference)
import numpy as np
import jax
import jax.numpy as jnp
from jax import lax
from jax.experimental import pallas as pl
from jax.experimental.pallas import tpu as pltpu

_PAIRS = [(0, 1), (0, 2), (0, 3), (1, 2), (1, 3), (2, 3)]


def _gram_constants():
    Sd = np.zeros((16, 64), dtype=np.float32)
    for n in range(4):
        for k in range(16):
            Sd[n * 4 + n, n * 16 + k] = 1.0
    To1 = np.zeros((96, 64), dtype=np.float32)
    To2 = np.zeros((96, 64), dtype=np.float32)
    So = np.zeros((16, 96), dtype=np.float32)
    for p, (n, m) in enumerate(_PAIRS):
        for k in range(16):
            To1[p * 16 + k, n * 16 + k] = 1.0
            To2[p * 16 + k, m * 16 + k] = 1.0
            So[n * 4 + m, p * 16 + k] = 1.0
            So[m * 4 + n, p * 16 + k] = 1.0
    return Sd, To1, To2, So


_SD_NP, _TO1_NP, _TO2_NP, _SO_NP = _gram_constants()
_TILE4 = {n: np.tile(np.eye(n, dtype=np.float32), (4, 1))
          for n in (3, 6, 16, 24, 32)}


def _bdmask(shape, rb, cb):
    ii = lax.broadcasted_iota(jnp.int32, shape, 0)
    jj = lax.broadcasted_iota(jnp.int32, shape, 1)
    return (ii // rb) == (jj // cb)


def _dgt(a, b):
    return lax.dot_general(a, b, (((1,), (1,)), ((), ())),
                           preferred_element_type=jnp.float32)


def _kron4(M, Ta, Tb):
    a, b = M.shape
    tiled = _dgt(jnp.dot(Ta, M, preferred_element_type=jnp.float32), Tb)
    return jnp.where(_bdmask((4 * a, 4 * b), a, b), tiled, 0.0)


def _kron4t(W, Ta, Tb):
    kin, kout = W.shape
    trw = jnp.dot(Ta, W, preferred_element_type=jnp.float32)
    tiled = _dgt(Tb, trw)
    return jnp.where(_bdmask((4 * kout, 4 * kin), kout, kin), tiled, 0.0)


def _bcol(Tn, b):
    return _dgt(Tn, b)


def _fused_kernel(spv_ref, sh_ref, level_ref, spr_ref, eps_ref,
                  W1_ref, W2_ref, Wc1_ref, Wmu_ref, Wls_ref,
                  Wsv1_ref, Wsv2_ref, Wsr1_ref, Wsr2_ref,
                  b1_ref, b2_ref, bc1_ref, bmu_ref, bls_ref,
                  bsv1_ref, bsv2_ref, bsr1_ref, bsr2_ref,
                  T3_ref, T6_ref, T16_ref, T24_ref, T32_ref,
                  Sd_ref, So_ref,
                  xspv_ref, xspr_ref, adj_ref,
                  Wa_s, Wenc_s, We_s, Wf_s,
                  ba_s, benc_s, be_s, bf_s):
    f32 = jnp.float32
    bf = jnp.bfloat16

    @pl.when(pl.program_id(0) == 0)
    def _pack():
        T3, T6, T16 = T3_ref[:], T6_ref[:], T16_ref[:]
        T24, T32 = T24_ref[:], T32_ref[:]
        Wa_s[:] = _kron4t(W1_ref[:], T3, T6).astype(bf)
        c1mu = _dgt(Wc1_ref[:], Wmu_ref[:])
        c1ls = _dgt(Wc1_ref[:], Wls_ref[:])
        Amu = jnp.dot(W2_ref[:], c1mu, preferred_element_type=f32)
        Als = jnp.dot(W2_ref[:], c1ls, preferred_element_type=f32)
        Wenc_s[:64, :] = _kron4t(Amu, T6, T16).astype(bf)
        Wenc_s[64:, :] = _kron4t(Als, T6, T16).astype(bf)
        t = jnp.dot(b2_ref[:], Wc1_ref[:],
                    preferred_element_type=f32) + bc1_ref[:]
        benc_s[:64, :] = _bcol(T16, _dgt(t, Wmu_ref[:]) + bmu_ref[:])
        benc_s[64:, :] = _bcol(T16, _dgt(t, Wls_ref[:]) + bls_ref[:])
        We_s[:96, :] = _kron4t(Wsv1_ref[:], T16, T24).astype(bf)
        We_s[96:, :] = _kron4t(Wsr1_ref[:], T16, T24).astype(bf)
        zq = jnp.zeros((64, 96), bf)
        Wf_s[:64, :96] = _kron4(Wsv2_ref[:], T16, T24).astype(bf)
        Wf_s[:64, 96:] = zq
        Wf_s[64:, :96] = zq
        Wf_s[64:, 96:] = _kron4(Wsr2_ref[:], T16, T24).astype(bf)
        ba_s[:] = _bcol(T6, b1_ref[:])
        be_s[:96, :] = _bcol(T24, bsv1_ref[:])
        be_s[96:, :] = _bcol(T24, bsr1_ref[:])
        bf_s[:64, :] = _bcol(T16, bsv2_ref[:])
        bf_s[64:, :] = _bcol(T16, bsr2_ref[:])

    dot = lambda a, b: jnp.dot(a, b.astype(bf), preferred_element_type=f32)
    x = jnp.concatenate(
        [spv_ref[:], sh_ref[:], level_ref[:], spr_ref[:]], axis=0)
    h1 = jnp.maximum(dot(Wa_s[:], x) + ba_s[:], 0.0)
    ml = dot(Wenc_s[:], h1) + benc_s[:]
    z = ml[:64, :] + eps_ref[:] * jnp.exp(ml[64:, :])

    z0, z1, z2, z3 = z[0:16], z[16:32], z[32:48], z[48:64]
    p1 = jnp.concatenate([z0, z0, z0, z1, z1, z2], axis=0)
    p2 = jnp.concatenate([z1, z2, z3, z2, z3, z3], axis=0)
    diag = dot(Sd_ref[:], z * z)
    adj_ref[:] = jax.nn.sigmoid(diag + dot(So_ref[:], p1 * p2))

    dh = jnp.maximum(dot(We_s[:], z) + be_s[:], 0.0)
    out = dot(Wf_s[:], dh) + bf_s[:]
    xspv_ref[:] = out[:64, :]
    xspr_ref[:] = out[64:, :]


def kernel(dev_sh, dev_spv, dev_spr, dev_level, sh_W1, sh_b1, sh_W2, sh_b2,
           c1_W, c1_b, cmu_W, cmu_b, cls_W, cls_b, spv_W1, spv_b1, spv_W2,
           spv_b2, spr_W1, spr_b1, spr_W2, spr_b2, eps):
    B = dev_sh.shape[0]
    cblk = 16384 if B % 16384 == 0 else B
    f32 = jnp.float32
    bf16 = jnp.bfloat16

    epsT = jnp.transpose(eps, (1, 2, 0)).reshape(64, B)
    T3, T6, T16, T24, T32 = (jnp.asarray(_TILE4[n]) for n in (3, 6, 16, 24, 32))
    Sd, So = jnp.asarray(_SD_NP, bf16), jnp.asarray(_SO_NP, bf16)

    col_spec = lambda r: pl.BlockSpec((r, cblk), lambda i: (0, i))
    full = lambda a: pl.BlockSpec(a.shape, lambda i: (0,) * a.ndim)
    scr = [pltpu.VMEM(s, bf16) for s in
           [(24, 12), (128, 24), (192, 64), (128, 192)]]
    scr += [pltpu.VMEM(s, f32) for s in
            [(24, 1), (128, 1), (192, 1), (128, 1)]]

    smalls = [sh_W1, sh_W2, c1_W, cmu_W.T, cls_W.T, spv_W1, spv_W2.T,
              spr_W1, spr_W2.T,
              sh_b1[None, :], sh_b2[None, :], c1_b[None, :], cmu_b[None, :],
              cls_b[None, :], spv_b1[None, :], spv_b2[None, :],
              spr_b1[None, :], spr_b2[None, :],
              T3, T6, T16, T24, T32, Sd, So]

    xspv, xspr, adj = pl.pallas_call(
        _fused_kernel,
        grid=(B // cblk,),
        in_specs=[col_spec(3), col_spec(3), col_spec(3), col_spec(3),
                  col_spec(64)] + [full(a) for a in smalls],
        out_specs=[col_spec(64), col_spec(64), col_spec(16)],
        out_shape=[jax.ShapeDtypeStruct((64, B), f32),
                   jax.ShapeDtypeStruct((64, B), f32),
                   jax.ShapeDtypeStruct((16, B), f32)],
        scratch_shapes=scr,
        compiler_params=pltpu.CompilerParams(
            dimension_semantics=("arbitrary",)),
    )(dev_spv.T, dev_sh.T, dev_level.T, dev_spr.T, epsT, *smalls)

    return (jnp.transpose(xspv.reshape(4, 16, B), (2, 0, 1)),
            jnp.transpose(xspr.reshape(4, 16, B), (2, 0, 1)),
            jnp.transpose(adj.reshape(4, 4, B), (2, 0, 1)))


# speedup vs baseline: 1.1632x; 1.1632x over previous
import numpy as np
import jax
import jax.numpy as jnp
from jax import lax
from jax.experimental import pallas as pl
from jax.experimental.pallas import tpu as pltpu

_PAIRS = [(0, 1), (0, 2), (0, 3), (1, 2), (1, 3), (2, 3)]


def _gram_constants():
    Sd = np.zeros((16, 64), dtype=np.float32)
    for n in range(4):
        for k in range(16):
            Sd[n * 4 + n, n * 16 + k] = 1.0
    To1 = np.zeros((96, 64), dtype=np.float32)
    To2 = np.zeros((96, 64), dtype=np.float32)
    So = np.zeros((16, 96), dtype=np.float32)
    for p, (n, m) in enumerate(_PAIRS):
        for k in range(16):
            To1[p * 16 + k, n * 16 + k] = 1.0
            To2[p * 16 + k, m * 16 + k] = 1.0
            So[n * 4 + m, p * 16 + k] = 1.0
            So[m * 4 + n, p * 16 + k] = 1.0
    return Sd, To1, To2, So


_SD_NP, _TO1_NP, _TO2_NP, _SO_NP = _gram_constants()
_TILE4 = {n: np.tile(np.eye(n, dtype=np.float32), (4, 1))
          for n in (3, 6, 16, 24, 32)}


def _bdmask(shape, rb, cb):
    ii = lax.broadcasted_iota(jnp.int32, shape, 0)
    jj = lax.broadcasted_iota(jnp.int32, shape, 1)
    return (ii // rb) == (jj // cb)


def _dgt(a, b):
    return lax.dot_general(a, b, (((1,), (1,)), ((), ())),
                           preferred_element_type=jnp.float32)


def _kron4(M, Ta, Tb):
    a, b = M.shape
    tiled = _dgt(jnp.dot(Ta, M, preferred_element_type=jnp.float32), Tb)
    return jnp.where(_bdmask((4 * a, 4 * b), a, b), tiled, 0.0)


def _kron4t(W, Ta, Tb):
    kin, kout = W.shape
    trw = jnp.dot(Ta, W, preferred_element_type=jnp.float32)
    tiled = _dgt(Tb, trw)
    return jnp.where(_bdmask((4 * kout, 4 * kin), kout, kin), tiled, 0.0)


def _bcol(Tn, b):
    return _dgt(Tn, b)


def _fused_kernel(spv_ref, sh_ref, level_ref, spr_ref, eps_ref,
                  W1_ref, W2_ref, Wc1_ref, Wmu_ref, Wls_ref,
                  Wsv1_ref, Wsv2_ref, Wsr1_ref, Wsr2_ref,
                  b1_ref, b2_ref, bc1_ref, bmu_ref, bls_ref,
                  bsv1_ref, bsv2_ref, bsr1_ref, bsr2_ref,
                  T3_ref, T6_ref, T16_ref, T24_ref, T32_ref,
                  Sd_ref, So_ref,
                  xspv_ref, xspr_ref, adj_ref,
                  Wa_s, Wenc_s, We_s, Wf_s,
                  ba_s, benc_s, be_s, bf_s):
    f32 = jnp.float32
    bf = jnp.bfloat16

    @pl.when(pl.program_id(0) == 0)
    def _pack():
        T3, T6, T16 = T3_ref[:], T6_ref[:], T16_ref[:]
        T24, T32 = T24_ref[:], T32_ref[:]
        Wa_s[:] = _kron4t(W1_ref[:], T3, T6).astype(bf)
        c1mu = _dgt(Wc1_ref[:], Wmu_ref[:])
        c1ls = _dgt(Wc1_ref[:], Wls_ref[:])
        Amu = jnp.dot(W2_ref[:], c1mu, preferred_element_type=f32)
        Als = jnp.dot(W2_ref[:], c1ls, preferred_element_type=f32)
        Wenc_s[:64, :] = _kron4t(Amu, T6, T16).astype(bf)
        Wenc_s[64:, :] = _kron4t(Als, T6, T16).astype(bf)
        t = jnp.dot(b2_ref[:], Wc1_ref[:],
                    preferred_element_type=f32) + bc1_ref[:]
        benc_s[:64, :] = _bcol(T16, _dgt(t, Wmu_ref[:]) + bmu_ref[:])
        benc_s[64:, :] = _bcol(T16, _dgt(t, Wls_ref[:]) + bls_ref[:])
        We_s[:96, :] = _kron4t(Wsv1_ref[:], T16, T24).astype(bf)
        We_s[96:, :] = _kron4t(Wsr1_ref[:], T16, T24).astype(bf)
        zq = jnp.zeros((64, 96), bf)
        Wf_s[:64, :96] = _kron4(Wsv2_ref[:], T16, T24).astype(bf)
        Wf_s[:64, 96:] = zq
        Wf_s[64:, :96] = zq
        Wf_s[64:, 96:] = _kron4(Wsr2_ref[:], T16, T24).astype(bf)
        ba_s[:] = _bcol(T6, b1_ref[:])
        be_s[:96, :] = _bcol(T24, bsv1_ref[:])
        be_s[96:, :] = _bcol(T24, bsr1_ref[:])
        bf_s[:64, :] = _bcol(T16, bsv2_ref[:])
        bf_s[64:, :] = _bcol(T16, bsr2_ref[:])

    dot = lambda a, b: jnp.dot(a, b.astype(bf), preferred_element_type=f32)
    x = jnp.concatenate(
        [spv_ref[:], sh_ref[:], level_ref[:], spr_ref[:]], axis=0)
    h1 = jnp.maximum(dot(Wa_s[:], x) + ba_s[:], 0.0)
    ml = dot(Wenc_s[:], h1) + benc_s[:]
    z = ml[:64, :] + eps_ref[:] * jnp.exp(ml[64:, :])

    z0, z1, z2, z3 = z[0:16], z[16:32], z[32:48], z[48:64]
    p1 = jnp.concatenate([z0, z0, z0, z1, z1, z2], axis=0)
    p2 = jnp.concatenate([z1, z2, z3, z2, z3, z3], axis=0)
    diag = dot(Sd_ref[:], z * z)
    adjf = jax.nn.sigmoid(diag + dot(So_ref[:], p1 * p2))
    adj_ref[:] = adjf.reshape(adj_ref.shape)

    dh = jnp.maximum(dot(We_s[:], z) + be_s[:], 0.0)
    out = dot(Wf_s[:], dh) + bf_s[:]
    xspv_ref[:] = out[:64, :]
    xspr_ref[:] = out[64:, :]


def kernel(dev_sh, dev_spv, dev_spr, dev_level, sh_W1, sh_b1, sh_W2, sh_b2,
           c1_W, c1_b, cmu_W, cmu_b, cls_W, cls_b, spv_W1, spv_b1, spv_W2,
           spv_b2, spr_W1, spr_b1, spr_W2, spr_b2, eps):
    B = dev_sh.shape[0]
    cblk = 8192 if B % 8192 == 0 else B
    f32 = jnp.float32
    bf16 = jnp.bfloat16

    epsT = jnp.transpose(eps, (1, 2, 0)).reshape(64, B)
    T3, T6, T16, T24, T32 = (jnp.asarray(_TILE4[n]) for n in (3, 6, 16, 24, 32))
    Sd, So = jnp.asarray(_SD_NP, bf16), jnp.asarray(_SO_NP, bf16)

    col_spec = lambda r: pl.BlockSpec((r, cblk), lambda i: (0, i))
    full = lambda a: pl.BlockSpec(a.shape, lambda i: (0,) * a.ndim)
    scr = [pltpu.VMEM(s, bf16) for s in
           [(24, 12), (128, 24), (192, 64), (128, 192)]]
    scr += [pltpu.VMEM(s, f32) for s in
            [(24, 1), (128, 1), (192, 1), (128, 1)]]

    smalls = [sh_W1, sh_W2, c1_W, cmu_W.T, cls_W.T, spv_W1, spv_W2.T,
              spr_W1, spr_W2.T,
              sh_b1[None, :], sh_b2[None, :], c1_b[None, :], cmu_b[None, :],
              cls_b[None, :], spv_b1[None, :], spv_b2[None, :],
              spr_b1[None, :], spr_b2[None, :],
              T3, T6, T16, T24, T32, Sd, So]

    xspv, xspr, adj = pl.pallas_call(
        _fused_kernel,
        grid=(B // cblk,),
        in_specs=[col_spec(3), col_spec(3), col_spec(3), col_spec(3),
                  col_spec(64)] + [full(a) for a in smalls],
        out_specs=[col_spec(64), col_spec(64),
                   pl.BlockSpec((4, 4, cblk), lambda i: (0, 0, i))],
        out_shape=[jax.ShapeDtypeStruct((64, B), f32),
                   jax.ShapeDtypeStruct((64, B), f32),
                   jax.ShapeDtypeStruct((4, 4, B), f32)],
        scratch_shapes=scr,
        compiler_params=pltpu.CompilerParams(
            dimension_semantics=("arbitrary",)),
    )(dev_spv.T, dev_sh.T, dev_level.T, dev_spr.T, epsT, *smalls)

    return (jnp.transpose(xspv.reshape(4, 16, B), (2, 0, 1)),
            jnp.transpose(xspr.reshape(4, 16, B), (2, 0, 1)),
            jnp.transpose(adj, (2, 0, 1)))
